# baseline (speedup 1.0000x reference)
import functools

import jax
import jax.numpy as jnp
from jax import lax
from jax.experimental import pallas as pl
from jax.experimental.pallas import tpu as pltpu
from jax.experimental.pallas import tpu_sc as plsc

_WD = 12800
_W = 640
_NTILES = 32
_BLK = 2048
_PADC = 16


def _deg_body(pt, n_win, w, dst_hbm, z_hbm, deg_out, deg_s, ones_v):
    cid = lax.axis_index("core")
    sid = lax.axis_index("subcore")
    pltpu.sync_copy(z_hbm, deg_s.at[pl.ds(sid * pt, pt)])

    @pl.loop(0, w, step=16)
    def _(i):
        ones_v[pl.ds(i, 16)] = jnp.full((16,), 1.0, jnp.float32)

    plsc.subcore_barrier()

    def body(i_vmem):
        pltpu.sync_copy(ones_v, deg_s.at[i_vmem.at[0]], add=True)

    pltpu.emit_pipeline(
        body,
        grid=(n_win,),
        in_specs=[pl.BlockSpec((1, w), lambda i: (0, i))],
        core_axis_name=("core", "subcore"),
        dimension_semantics=(pltpu.PARALLEL,),
    )(dst_hbm)
    plsc.subcore_barrier()
    pltpu.sync_copy(deg_s.at[pl.ds(sid * pt, pt)],
                    deg_out.at[cid, 0, pl.ds(sid * pt, pt)])


def _agg_body(pt, rounds, w, src_hbm, dst_hbm, y_hbm, z_hbm, s_out,
              s_spmem, rows_v, isrc_v, idst_v, sem_i, sem_s, sem_g):
    cid = lax.axis_index("core")
    sid = lax.axis_index("subcore")
    wid = sid * 2 + cid

    def win_off(r):
        return (r * _NTILES + wid) * w

    def issue_idx(r, slot):
        off = win_off(r)
        pltpu.async_copy(src_hbm.at[pl.ds(off, w)], isrc_v.at[slot],
                         sem_i.at[slot])
        pltpu.async_copy(dst_hbm.at[pl.ds(off, w)], idst_v.at[slot],
                         sem_i.at[slot])

    def wait_idx(r, slot):
        off = win_off(r)
        pltpu.make_async_copy(src_hbm.at[pl.ds(off, w)], isrc_v.at[slot],
                              sem_i.at[slot]).wait()
        pltpu.make_async_copy(dst_hbm.at[pl.ds(off, w)], idst_v.at[slot],
                              sem_i.at[slot]).wait()

    def wait_scatter(rows_slot, idx_slot):
        pltpu.make_async_copy(rows_v.at[rows_slot],
                              s_spmem.at[idst_v.at[idx_slot]],
                              sem_s.at[rows_slot]).wait()

    issue_idx(0, 0)
    issue_idx(1, 1)
    pltpu.sync_copy(z_hbm, s_spmem.at[pl.ds(sid * pt, pt)])
    plsc.subcore_barrier()

    @pl.loop(0, rounds, step=4)
    def _(r0):
        for j in range(4):
            r = r0 + j
            si = j
            sr = j % 2

            @pl.when(r >= 2)
            def _():
                wait_scatter(sr, (j + 2) % 4)

            @pl.when(r + 2 < rounds)
            def _():
                issue_idx(r + 2, (j + 2) % 4)

            wait_idx(r, si)
            pltpu.async_copy(y_hbm.at[isrc_v.at[si]], rows_v.at[sr],
                             sem_g).wait()
            pltpu.async_copy(rows_v.at[sr], s_spmem.at[idst_v.at[si]],
                             sem_s.at[sr], add=True)

    wait_scatter(0, (rounds - 2) % 4)
    wait_scatter(1, (rounds - 1) % 4)
    plsc.subcore_barrier()
    pltpu.sync_copy(s_spmem.at[pl.ds(sid * pt, pt)],
                    s_out.at[cid, pl.ds(sid * pt, pt)])


def _y_body(x_ref, w_ref, deg_ref, y_ref):
    xw = jnp.dot(x_ref[...], w_ref[...], preferred_element_type=jnp.float32)
    deg = deg_ref[0, 0] + deg_ref[1, 0] + 1.0
    dis = lax.rsqrt(deg)
    blk = xw.shape[0]
    pad = jnp.zeros((blk, _PADC - xw.shape[1] - 1), jnp.float32)
    y_ref[...] = jnp.concatenate([xw * dis[:, None], dis[:, None], pad], axis=1)


def _out_body(d_hid, sp_ref, y_ref, bg_ref, wf_ref, bf_ref, hid_ref, out_ref):
    s = sp_ref[0] + sp_ref[1]
    y = y_ref[...]
    t = s[:, :d_hid] + y[:, :d_hid]
    dis = y[:, d_hid:d_hid + 1]
    hidden = dis * t + bg_ref[...]
    hid_ref[...] = hidden
    out_ref[...] = jnp.maximum(
        jnp.dot(hidden, wf_ref[...], preferred_element_type=jnp.float32)
        + bf_ref[...], 0.0)


def kernel(x, edge_index, W_gcn, b_gcn, W_fc, b_fc):
    n, d_in = x.shape
    d_hid = W_gcn.shape[1]
    d_out = W_fc.shape[1]
    e = edge_index.shape[1]

    mesh = plsc.VectorSubcoreMesh(core_axis_name="core",
                                  subcore_axis_name="subcore")
    sc_params = pltpu.CompilerParams(use_tc_tiling_on_sc=False)
    nc, nsub = 2, 16
    pt = -(-n // nsub)
    pt = (pt + 127) // 128 * 128
    npad = pt * nsub
    assert npad % _BLK == 0

    rounds = -(-e // (_NTILES * _W))
    rounds = (rounds + 3) // 4 * 4
    epad = rounds * _NTILES * _W
    assert epad % _WD == 0
    src = edge_index[0].astype(jnp.int32)
    dst = edge_index[1].astype(jnp.int32)
    src = jnp.concatenate([src, jnp.zeros((epad - e,), jnp.int32)])
    dst = jnp.concatenate([dst, jnp.full((epad - e,), npad - 1, jnp.int32)])

    z1 = jnp.zeros((pt,), jnp.float32)
    z16 = jnp.zeros((pt, _PADC), jnp.float32)

    deg_parts = pl.kernel(
        functools.partial(_deg_body, pt, epad // _WD, _WD),
        out_type=jax.ShapeDtypeStruct((nc, 1, npad), jnp.float32),
        mesh=mesh,
        scratch_types=[pltpu.VMEM_SHARED((npad,), jnp.float32),
                       pltpu.VMEM((_WD,), jnp.float32)],
        compiler_params=sc_params,
    )(dst.reshape(1, epad), z1)

    y = pl.pallas_call(
        _y_body,
        grid=(npad // _BLK,),
        in_specs=[pl.BlockSpec((_BLK, d_in), lambda i: (i, 0)),
                  pl.BlockSpec((d_in, d_hid), lambda i: (0, 0)),
                  pl.BlockSpec((nc, 1, _BLK), lambda i: (0, 0, i))],
        out_specs=pl.BlockSpec((_BLK, _PADC), lambda i: (i, 0)),
        out_shape=jax.ShapeDtypeStruct((n, _PADC), jnp.float32),
    )(x, W_gcn, deg_parts)

    s_parts = pl.kernel(
        functools.partial(_agg_body, pt, rounds, _W),
        out_type=jax.ShapeDtypeStruct((nc, npad, _PADC), jnp.float32),
        mesh=mesh,
        scratch_types=[pltpu.VMEM_SHARED((npad, _PADC), jnp.float32),
                       pltpu.VMEM((2, _W, _PADC), jnp.float32),
                       pltpu.VMEM((4, _W), jnp.int32),
                       pltpu.VMEM((4, _W), jnp.int32),
                       pltpu.SemaphoreType.DMA((4,)),
                       pltpu.SemaphoreType.DMA((2,)),
                       pltpu.SemaphoreType.DMA],
        compiler_params=sc_params,
    )(src, dst, y, z16)

    hidden, out = pl.pallas_call(
        functools.partial(_out_body, d_hid),
        grid=(npad // _BLK,),
        in_specs=[pl.BlockSpec((nc, _BLK, _PADC), lambda i: (0, i, 0)),
                  pl.BlockSpec((_BLK, _PADC), lambda i: (i, 0)),
                  pl.BlockSpec((1, d_hid), lambda i: (0, 0)),
                  pl.BlockSpec((d_hid, d_out), lambda i: (0, 0)),
                  pl.BlockSpec((1, d_out), lambda i: (0, 0))],
        out_specs=[pl.BlockSpec((_BLK, d_hid), lambda i: (i, 0)),
                   pl.BlockSpec((_BLK, d_out), lambda i: (i, 0))],
        out_shape=[jax.ShapeDtypeStruct((n, d_hid), jnp.float32),
                   jax.ShapeDtypeStruct((n, d_out), jnp.float32)],
    )(s_parts, y, b_gcn.reshape(1, d_hid), W_fc, b_fc.reshape(1, d_out))

    return (hidden, out)


# speedup vs baseline: 1.5365x; 1.5365x over previous
import functools

import jax
import jax.numpy as jnp
from jax import lax
from jax.experimental import pallas as pl
from jax.experimental.pallas import tpu as pltpu
from jax.experimental.pallas import tpu_sc as plsc

_W = 1280
_BLK = 2048
_PADC = 16
_LANES = 128


def _deg_body(pt, n_win, w, dst_hbm, z_hbm, deg0_out, deg1_out, deg_s, ones_v):
    cid = lax.axis_index("core")
    sid = lax.axis_index("subcore")
    pltpu.sync_copy(z_hbm, deg_s.at[pl.ds(sid * pt, pt)])

    @pl.loop(0, w, step=16)
    def _(i):
        ones_v[pl.ds(i, 16)] = jnp.full((16,), 1.0, jnp.float32)

    plsc.subcore_barrier()

    def body(i_vmem):
        pltpu.sync_copy(ones_v, deg_s.at[i_vmem.at[0]], add=True)

    pltpu.emit_pipeline(
        body,
        grid=(n_win,),
        in_specs=[pl.BlockSpec((1, w), lambda i: (0, i))],
        core_axis_name=("core", "subcore"),
        dimension_semantics=(pltpu.PARALLEL,),
    )(dst_hbm)
    plsc.subcore_barrier()

    @pl.when(cid == 0)
    def _():
        pltpu.sync_copy(deg_s.at[pl.ds(sid * pt, pt)],
                        deg0_out.at[pl.ds(sid * pt, pt)])

    @pl.when(cid == 1)
    def _():
        pltpu.sync_copy(deg_s.at[pl.ds(sid * pt, pt)],
                        deg1_out.at[pl.ds(sid * pt, pt)])


def _agg_body(pt, n_win, w, src_hbm, dst_hbm, y_hbm, z_hbm, s_out,
              s_spmem, rows_v):
    cid = lax.axis_index("core")
    sid = lax.axis_index("subcore")
    pltpu.sync_copy(z_hbm, s_spmem.at[pl.ds(sid * pt, pt)])
    plsc.subcore_barrier()

    def body(s_vmem, d_vmem):
        pltpu.sync_copy(y_hbm.at[s_vmem.at[0]], rows_v)
        pltpu.sync_copy(rows_v, s_spmem.at[d_vmem.at[0]], add=True)
    pltpu.emit_pipeline(
        body,
        grid=(n_win,),
        in_specs=[pl.BlockSpec((1, w), lambda i: (0, i)),
                  pl.BlockSpec((1, w), lambda i: (0, i))],
        core_axis_name=("core", "subcore"),
        dimension_semantics=(pltpu.PARALLEL,),
    )(src_hbm, dst_hbm)
    plsc.subcore_barrier()
    pltpu.sync_copy(s_spmem.at[pl.ds(sid * pt, pt)],
                    s_out.at[cid, pl.ds(sid * pt, pt), pl.ds(0, _PADC)])


def _y_body(x_ref, w_ref, d0_ref, d1_ref, y_ref):
    xw = jnp.dot(x_ref[...], w_ref[...], preferred_element_type=jnp.float32)
    blk = xw.shape[0]
    deg = (d0_ref[...] + d1_ref[...] + 1.0).reshape(blk, 1)
    dis = lax.rsqrt(deg)
    pad = jnp.zeros((blk, _LANES - xw.shape[1] - 1), jnp.float32)
    y_ref[...] = jnp.concatenate([xw * dis, dis, pad], axis=1)


def _out_body(d_hid, sp_ref, y_ref, bg_ref, wf_ref, bf_ref, hid_ref, out_ref):
    s = sp_ref[0] + sp_ref[1]
    y = y_ref[...]
    t = s[:, :d_hid] + y[:, :d_hid]
    dis = y[:, d_hid:d_hid + 1]
    hidden = dis * t + bg_ref[...]
    hid_ref[...] = hidden
    out_ref[...] = jnp.maximum(
        jnp.dot(hidden, wf_ref[...], preferred_element_type=jnp.float32)
        + bf_ref[...], 0.0)


def kernel(x, edge_index, W_gcn, b_gcn, W_fc, b_fc):
    n, d_in = x.shape
    d_hid = W_gcn.shape[1]
    d_out = W_fc.shape[1]
    e = edge_index.shape[1]
    assert e % _W == 0

    mesh = plsc.VectorSubcoreMesh(core_axis_name="core",
                                  subcore_axis_name="subcore")
    sc_params = pltpu.CompilerParams(use_tc_tiling_on_sc=False)
    nc, nsub = 2, 16
    pt = -(-n // nsub)
    pt = (pt + 127) // 128 * 128
    npad = pt * nsub
    assert npad % _BLK == 0

    src = (edge_index[0].astype(jnp.int32) * 8).reshape(1, e)
    dst = edge_index[1].astype(jnp.int32).reshape(1, e)
    z1 = jnp.zeros((pt,), jnp.float32)
    z16 = jnp.zeros((pt, _PADC), jnp.float32)

    deg0, deg1 = pl.kernel(
        functools.partial(_deg_body, pt, e // _W, _W),
        out_type=[jax.ShapeDtypeStruct((npad,), jnp.float32),
                  jax.ShapeDtypeStruct((npad,), jnp.float32)],
        mesh=mesh,
        scratch_types=[pltpu.VMEM_SHARED((npad,), jnp.float32),
                       pltpu.VMEM((_W,), jnp.float32)],
        compiler_params=sc_params,
    )(dst, z1)

    y = pl.pallas_call(
        _y_body,
        grid=(npad // _BLK,),
        in_specs=[pl.BlockSpec((_BLK, d_in), lambda i: (i, 0)),
                  pl.BlockSpec((d_in, d_hid), lambda i: (0, 0)),
                  pl.BlockSpec((_BLK,), lambda i: (i,)),
                  pl.BlockSpec((_BLK,), lambda i: (i,))],
        out_specs=pl.BlockSpec((_BLK, _LANES), lambda i: (i, 0)),
        out_shape=jax.ShapeDtypeStruct((npad, _LANES), jnp.float32),
    )(x, W_gcn, deg0, deg1)

    s_parts = pl.kernel(
        functools.partial(_agg_body, pt, e // _W, _W),
        out_type=jax.ShapeDtypeStruct((nc, npad, _LANES), jnp.float32),
        mesh=mesh,
        scratch_types=[pltpu.VMEM_SHARED((npad, _PADC), jnp.float32),
                       pltpu.VMEM((_W, _PADC), jnp.float32)],
        compiler_params=sc_params,
    )(src, dst, y.reshape(npad * 8, _PADC), z16)

    hidden, out = pl.pallas_call(
        functools.partial(_out_body, d_hid),
        grid=(npad // _BLK,),
        in_specs=[pl.BlockSpec((nc, _BLK, _LANES), lambda i: (0, i, 0)),
                  pl.BlockSpec((_BLK, _LANES), lambda i: (i, 0)),
                  pl.BlockSpec((1, d_hid), lambda i: (0, 0)),
                  pl.BlockSpec((d_hid, d_out), lambda i: (0, 0)),
                  pl.BlockSpec((1, d_out), lambda i: (0, 0))],
        out_specs=[pl.BlockSpec((_BLK, d_hid), lambda i: (i, 0)),
                   pl.BlockSpec((_BLK, d_out), lambda i: (i, 0))],
        out_shape=[jax.ShapeDtypeStruct((n, d_hid), jnp.float32),
                   jax.ShapeDtypeStruct((n, d_out), jnp.float32)],
    )(s_parts, y, b_gcn.reshape(1, d_hid), W_fc, b_fc.reshape(1, d_out))

    return (hidden, out)
